# baseline (speedup 1.0000x reference)
import jax
import jax.numpy as jnp
from jax.experimental import pallas as pl
from jax.experimental.pallas import tpu as pltpu

N = 10000
F = 128
ALPHA = 0.3

_BM = 400
_BM_BF = 1000
_BM_SMALL = 1000

_WNAMES = ['w1', 'w2', 'w3', 'w4', 'wll1', 'wuu1', 'wuu2', 'wll2',
           'w3m', 'w3s', 'wiu1', 'wiu2', 'w4m', 'w4s', 'wum1', 'wum2',
           'wus1', 'wus2', 'wim1', 'wim2', 'wis1', 'wis2']
_WOFF = {n: i * F for i, n in enumerate(_WNAMES)}
_NW = len(_WNAMES) * F

_BNAMES = ['bA', 'bB1', 'bB2', 'bC1', 'bC2', 'bD1', 'bD2', 'bD3',
           'bE1', 'bE2', 'buu', 'biu', 'bum', 'bus', 'bim', 'bis']
_BOFF = {n: i * F for i, n in enumerate(_BNAMES)}
_NB = len(_BNAMES) * F


def _leaky(x):
    return jnp.where(x >= 0, x, ALPHA * x)


def _bf(x):
    return x.astype(jnp.bfloat16)


def _dot(a, b):
    return jnp.dot(a, b, preferred_element_type=jnp.float32)


def _w(wall_ref, name, nblk=1):
    o = _WOFF[name]
    return wall_ref[:, o:o + nblk * F]


def _b(ball_ref, name, nblk=1):
    o = _BOFF[name]
    return ball_ref[:, o:o + nblk * F]


def _proj_body(x_ref, wallf_ref, rA_ref, wallbf_ref):
    @pl.when(pl.program_id(0) == 0)
    def _prep():
        wallbf_ref[...] = _bf(wallf_ref[...])

    rA_ref[...] = _bf(_dot(_bf(x_ref[...]), _bf(wallf_ref[:, :F])))


def _proj(x, wall_f32):
    return pl.pallas_call(
        _proj_body,
        grid=(N // _BM_SMALL,),
        in_specs=[pl.BlockSpec((_BM_SMALL, F), lambda i: (i, 0)),
                  pl.BlockSpec((F, _NW), lambda i: (0, 0))],
        out_specs=[pl.BlockSpec((_BM_SMALL, F), lambda i: (i, 0)),
                   pl.BlockSpec((F, _NW), lambda i: (0, 0))],
        out_shape=[jax.ShapeDtypeStruct((N, F), jnp.bfloat16),
                   jax.ShapeDtypeStruct((F, _NW), jnp.bfloat16)],
    )(x, wall_f32)


def _pass(body, adj, rhs, row_ins, res_ins, out_w, out_dt, bm):
    w = rhs.shape[1]
    in_specs = [pl.BlockSpec((bm, N), lambda i: (i, 0)),
                pl.BlockSpec((N, w), lambda i: (0, 0))]
    for a in row_ins:
        in_specs.append(pl.BlockSpec((bm, a.shape[1]), lambda i: (i, 0)))
    for a in res_ins:
        in_specs.append(pl.BlockSpec(
            tuple(a.shape), lambda i, n=len(a.shape): (0,) * n))
    out_specs = [pl.BlockSpec((bm, ww), lambda i: (i, 0)) for ww in out_w]
    out_shape = [jax.ShapeDtypeStruct((N, ww), dt)
                 for ww, dt in zip(out_w, out_dt)]
    return pl.pallas_call(
        body,
        grid=(N // bm,),
        in_specs=in_specs,
        out_specs=out_specs,
        out_shape=out_shape,
        compiler_params=pltpu.CompilerParams(
            dimension_semantics=("parallel",)),
    )(adj, rhs, *row_ins, *res_ins)


def _passA_body(adj_ref, rhs_ref, vfea_ref, wall_ref, ball_ref,
                abf_ref, rB_ref):
    a = _bf(adj_ref[...])
    abf_ref[...] = a
    uh1 = _leaky(_dot(a, rhs_ref[...]) + _b(ball_ref, 'bA'))
    rB_ref[...] = jnp.concatenate(
        [_bf(_dot(_bf(vfea_ref[...]), _w(wall_ref, 'w2'))),
         _bf(_dot(_bf(uh1), _w(wall_ref, 'w3')))], axis=1)


def _passB_body(adj_ref, rhs_ref, ufea_ref, wall_ref, ball_ref,
                rC_ref, u_ref):
    t = _leaky(_dot(_bf(adj_ref[...]), rhs_ref[...])
               + _b(ball_ref, 'bB1', 2))
    ih1, uh2 = t[:, :F], t[:, F:]
    u = jnp.maximum(_dot(_bf(uh2), _w(wall_ref, 'wuu1'))
                    + _dot(_bf(ufea_ref[...]), _w(wall_ref, 'wuu2'))
                    + _b(ball_ref, 'buu'), 0.0)
    u_ref[...] = u
    rC_ref[...] = jnp.concatenate(
        [_bf(_dot(_bf(ih1), _w(wall_ref, 'w4'))),
         _bf(_dot(_bf(u), _w(wall_ref, 'wll1')))], axis=1)


def _passC_body(adj_ref, rhs_ref, vfea_ref, wall_ref, ball_ref,
                rD_ref, v_ref):
    t = _leaky(_dot(adj_ref[...], rhs_ref[...]) + _b(ball_ref, 'bC1', 2))
    ih2, uhh = t[:, :F], t[:, F:]
    v = jnp.maximum(_dot(_bf(ih2), _w(wall_ref, 'wiu1'))
                    + _dot(_bf(vfea_ref[...]), _w(wall_ref, 'wiu2'))
                    + _b(ball_ref, 'biu'), 0.0)
    v_ref[...] = v
    rD_ref[...] = jnp.concatenate(
        [_bf(_dot(_bf(v), _w(wall_ref, 'wll2'))),
         _bf(_dot(_bf(uhh), _w(wall_ref, 'w3m', 2)))], axis=1)


def _passD_body(adj_ref, rhs_ref, u_ref, wall_ref, ball_ref,
                rE_ref, mu_ref, lu_ref):
    t = _leaky(_dot(_bf(adj_ref[...]), rhs_ref[...])
               + _b(ball_ref, 'bD1', 3))
    ihh, gmu, gsu = t[:, :F], t[:, F:2 * F], t[:, 2 * F:]
    rE_ref[...] = _bf(_dot(_bf(ihh), _w(wall_ref, 'w4m', 2)))
    ub = _bf(u_ref[...])
    mu_ref[...] = (_dot(_bf(gmu), _w(wall_ref, 'wum1'))
                   + _dot(ub, _w(wall_ref, 'wum2')) + _b(ball_ref, 'bum'))
    lu_ref[...] = (_dot(_bf(gsu), _w(wall_ref, 'wus1'))
                   + _dot(ub, _w(wall_ref, 'wus2')) + _b(ball_ref, 'bus'))


def _passE_body(adj_ref, rhs_ref, v_ref, wall_ref, ball_ref,
                mi_ref, li_ref):
    t = _leaky(_dot(adj_ref[...], rhs_ref[...]) + _b(ball_ref, 'bE1', 2))
    gmi, gsi = t[:, :F], t[:, F:]
    vb = _bf(v_ref[...])
    mi_ref[...] = (_dot(_bf(gmi), _w(wall_ref, 'wim1'))
                   + _dot(vb, _w(wall_ref, 'wim2')) + _b(ball_ref, 'bim'))
    li_ref[...] = (_dot(_bf(gsi), _w(wall_ref, 'wis1'))
                   + _dot(vb, _w(wall_ref, 'wis2')) + _b(ball_ref, 'bis'))


def kernel(ufea, vfea, UV_adj, VU_adj, params):
    p = params

    wall = jnp.concatenate(
        [p['l0_gc1_W'], p['l0_gc2_W'], p['l0_gc3_W'], p['l0_gc4_W'],
         p['ll_gc1_W'], p['l0_uu_W'][:F], p['l0_uu_W'][F:], p['ll_gc2_W'],
         p['ll_gc3m_W'], p['ll_gc3s_W'], p['l0_iu_W'][:F], p['l0_iu_W'][F:],
         p['ll_gc4m_W'], p['ll_gc4s_W'], p['ll_uum_W'][:F], p['ll_uum_W'][F:],
         p['ll_uus_W'][:F], p['ll_uus_W'][F:], p['ll_ium_W'][:F],
         p['ll_ium_W'][F:], p['ll_ius_W'][:F], p['ll_ius_W'][F:]], axis=1)
    ball = jnp.concatenate(
        [p['l0_gc1_b'], p['l0_gc2_b'], p['l0_gc3_b'], p['l0_gc4_b'],
         p['ll_gc1_b'], p['ll_gc2_b'], p['ll_gc3m_b'], p['ll_gc3s_b'],
         p['ll_gc4m_b'], p['ll_gc4s_b'], p['l0_uu_b'], p['l0_iu_b'],
         p['ll_uum_b'], p['ll_uus_b'], p['ll_ium_b'], p['ll_ius_b']])[None, :]

    rA, wall_bf = _proj(ufea, wall)

    vu_bf, rB = _pass(
        _passA_body, VU_adj, rA, [vfea], [wall_bf, ball],
        [N, 2 * F], [jnp.bfloat16, jnp.bfloat16], _BM)

    rC, u = _pass(
        _passB_body, UV_adj, rB, [ufea], [wall_bf, ball],
        [2 * F, F], [jnp.bfloat16, jnp.float32], _BM)

    rD, v = _pass(
        _passC_body, vu_bf, rC, [vfea], [wall_bf, ball],
        [3 * F, F], [jnp.bfloat16, jnp.float32], _BM_BF)

    rE, mean_u, logstd_u = _pass(
        _passD_body, UV_adj, rD, [u], [wall_bf, ball],
        [2 * F, F, F], [jnp.bfloat16, jnp.float32, jnp.float32], _BM)

    mean_i, logstd_i = _pass(
        _passE_body, vu_bf, rE, [v], [wall_bf, ball],
        [F, F], [jnp.float32, jnp.float32], _BM_BF)

    return (mean_u, mean_i, mean_u, mean_i, logstd_u, logstd_i)


# speedup vs baseline: 1.0563x; 1.0563x over previous
import jax
import jax.numpy as jnp
from jax.experimental import pallas as pl
from jax.experimental.pallas import tpu as pltpu

N = 10000
F = 128
ALPHA = 0.3

_BM = 400
_BM_BF = 1000
_BM_SMALL = 1000


def _leaky(x):
    return jnp.where(x >= 0, x, ALPHA * x)


def _bf(x):
    return x.astype(jnp.bfloat16)


def _dot(a, b):
    return jnp.dot(a, b, preferred_element_type=jnp.float32)


def _proj_body(x_ref, w_ref, o_ref):
    o_ref[...] = _bf(_dot(_bf(x_ref[...]), w_ref[...]))


def _proj(x, w_bf):
    return pl.pallas_call(
        _proj_body,
        grid=(N // _BM_SMALL,),
        in_specs=[pl.BlockSpec((_BM_SMALL, F), lambda i: (i, 0)),
                  pl.BlockSpec((F, F), lambda i: (0, 0))],
        out_specs=pl.BlockSpec((_BM_SMALL, F), lambda i: (i, 0)),
        out_shape=jax.ShapeDtypeStruct((N, F), jnp.bfloat16),
    )(x, w_bf)


def _pass(body, adj, rhs, bias, row_ins, res_ins, out_w, out_dt, bm):
    w = rhs.shape[1]
    in_specs = [pl.BlockSpec((bm, N), lambda i: (i, 0)),
                pl.BlockSpec((N, w), lambda i: (0, 0)),
                pl.BlockSpec((1, w), lambda i: (0, 0))]
    for a in row_ins:
        in_specs.append(pl.BlockSpec((bm, a.shape[1]), lambda i: (i, 0)))
    for a in res_ins:
        in_specs.append(pl.BlockSpec(
            tuple(a.shape), lambda i, n=len(a.shape): (0,) * n))
    out_specs = [pl.BlockSpec((bm, ww), lambda i: (i, 0)) for ww in out_w]
    out_shape = [jax.ShapeDtypeStruct((N, ww), dt)
                 for ww, dt in zip(out_w, out_dt)]
    return pl.pallas_call(
        body,
        grid=(N // bm,),
        in_specs=in_specs,
        out_specs=out_specs,
        out_shape=out_shape,
        compiler_params=pltpu.CompilerParams(
            dimension_semantics=("parallel",)),
    )(adj, rhs, bias, *row_ins, *res_ins)


def _gcn_tile(adj_ref, rhs_ref, b_ref):
    a = adj_ref[...]
    if a.dtype != jnp.bfloat16:
        a = _bf(a)
    return _leaky(_dot(a, rhs_ref[...]) + b_ref[...])


def _passA_body(adj_ref, rhs_ref, b_ref, vfea_ref, w2_ref, w3_ref,
                abf_ref, rB_ref):
    a = _bf(adj_ref[...])
    abf_ref[...] = a
    uh1 = _leaky(_dot(a, rhs_ref[...]) + b_ref[...])
    rB_ref[...] = jnp.concatenate(
        [_bf(_dot(_bf(vfea_ref[...]), w2_ref[...])),
         _bf(_dot(_bf(uh1), w3_ref[...]))], axis=1)


def _passB_body(adj_ref, rhs_ref, b_ref, ufea_ref,
                w4_ref, wll1_ref, wuu1_ref, wuu2_ref, buu_ref,
                rC_ref, u_ref):
    t = _gcn_tile(adj_ref, rhs_ref, b_ref)
    ih1, uh2 = t[:, :F], t[:, F:]
    u = jnp.maximum(_dot(_bf(uh2), wuu1_ref[...])
                    + _dot(_bf(ufea_ref[...]), wuu2_ref[...])
                    + buu_ref[...], 0.0)
    u_ref[...] = u
    rC_ref[...] = jnp.concatenate(
        [_bf(_dot(_bf(ih1), w4_ref[...])),
         _bf(_dot(_bf(u), wll1_ref[...]))], axis=1)


def _passC_body(adj_ref, rhs_ref, b_ref, vfea_ref,
                wll2_ref, w3ms_ref, wiu1_ref, wiu2_ref, biu_ref,
                rD_ref, v_ref):
    t = _gcn_tile(adj_ref, rhs_ref, b_ref)
    ih2, uhh = t[:, :F], t[:, F:]
    v = jnp.maximum(_dot(_bf(ih2), wiu1_ref[...])
                    + _dot(_bf(vfea_ref[...]), wiu2_ref[...])
                    + biu_ref[...], 0.0)
    v_ref[...] = v
    rD_ref[...] = jnp.concatenate(
        [_bf(_dot(_bf(v), wll2_ref[...])),
         _bf(_dot(_bf(uhh), w3ms_ref[...]))], axis=1)


def _passD_body(adj_ref, rhs_ref, b_ref, u_ref,
                w4ms_ref, wum1_ref, wum2_ref, bum_ref,
                wus1_ref, wus2_ref, bus_ref,
                rE_ref, mu_ref, lu_ref):
    t = _gcn_tile(adj_ref, rhs_ref, b_ref)
    ihh, gmu, gsu = t[:, :F], t[:, F:2 * F], t[:, 2 * F:]
    rE_ref[...] = _bf(_dot(_bf(ihh), w4ms_ref[...]))
    ub = _bf(u_ref[...])
    mu_ref[...] = (_dot(_bf(gmu), wum1_ref[...]) + _dot(ub, wum2_ref[...])
                   + bum_ref[...])
    lu_ref[...] = (_dot(_bf(gsu), wus1_ref[...]) + _dot(ub, wus2_ref[...])
                   + bus_ref[...])


def _passE_body(adj_ref, rhs_ref, b_ref, v_ref,
                wim1_ref, wim2_ref, bim_ref,
                wis1_ref, wis2_ref, bis_ref,
                mi_ref, li_ref):
    t = _gcn_tile(adj_ref, rhs_ref, b_ref)
    gmi, gsi = t[:, :F], t[:, F:]
    vb = _bf(v_ref[...])
    mi_ref[...] = (_dot(_bf(gmi), wim1_ref[...]) + _dot(vb, wim2_ref[...])
                   + bim_ref[...])
    li_ref[...] = (_dot(_bf(gsi), wis1_ref[...]) + _dot(vb, wis2_ref[...])
                   + bis_ref[...])


def kernel(ufea, vfea, UV_adj, VU_adj, params):
    p = params

    def wcat(*names):
        return _bf(jnp.concatenate([p[n] for n in names], axis=1))

    def bcat(*names):
        return jnp.concatenate([p[n] for n in names])[None, :]

    rA = _proj(ufea, _bf(p['l0_gc1_W']))

    vu_bf, rB = _pass(
        _passA_body, VU_adj, rA, p['l0_gc1_b'][None, :],
        [vfea], [_bf(p['l0_gc2_W']), _bf(p['l0_gc3_W'])],
        [N, 2 * F], [jnp.bfloat16, jnp.bfloat16], _BM)

    rC, u = _pass(
        _passB_body, UV_adj, rB, bcat('l0_gc2_b', 'l0_gc3_b'),
        [ufea],
        [_bf(p['l0_gc4_W']), _bf(p['ll_gc1_W']),
         _bf(p['l0_uu_W'][:F]), _bf(p['l0_uu_W'][F:]), p['l0_uu_b'][None, :]],
        [2 * F, F], [jnp.bfloat16, jnp.float32], _BM)

    rD, v = _pass(
        _passC_body, vu_bf, rC, bcat('l0_gc4_b', 'll_gc1_b'),
        [vfea],
        [_bf(p['ll_gc2_W']), wcat('ll_gc3m_W', 'll_gc3s_W'),
         _bf(p['l0_iu_W'][:F]), _bf(p['l0_iu_W'][F:]), p['l0_iu_b'][None, :]],
        [3 * F, F], [jnp.bfloat16, jnp.float32], _BM_BF)

    rE, mean_u, logstd_u = _pass(
        _passD_body, UV_adj, rD, bcat('ll_gc2_b', 'll_gc3m_b', 'll_gc3s_b'),
        [u],
        [wcat('ll_gc4m_W', 'll_gc4s_W'),
         _bf(p['ll_uum_W'][:F]), _bf(p['ll_uum_W'][F:]), p['ll_uum_b'][None, :],
         _bf(p['ll_uus_W'][:F]), _bf(p['ll_uus_W'][F:]), p['ll_uus_b'][None, :]],
        [2 * F, F, F], [jnp.bfloat16, jnp.float32, jnp.float32], _BM)

    mean_i, logstd_i = _pass(
        _passE_body, vu_bf, rE, bcat('ll_gc4m_b', 'll_gc4s_b'),
        [v],
        [_bf(p['ll_ium_W'][:F]), _bf(p['ll_ium_W'][F:]), p['ll_ium_b'][None, :],
         _bf(p['ll_ius_W'][:F]), _bf(p['ll_ius_W'][F:]), p['ll_ius_b'][None, :]],
        [F, F], [jnp.float32, jnp.float32], _BM_BF)

    return (mean_u, mean_i, mean_u, mean_i, logstd_u, logstd_i)
